# baseline (device time: 15879 ns/iter reference)
import jax
import jax.numpy as jnp
from jax import lax
from jax.experimental import pallas as pl
from jax.experimental.pallas import tpu as pltpu

CPQ = 4


def kernel(x):
    m_per, n = x.shape
    q = m_per // 4
    rows = q // CPQ

    def body(
        x_ref, out_ref, xbar, zbar,
        ysend, yrecv, x2send, x2recv, z2send, z2recv,
        x3send, x3recv, z3send, z3recv,
    ):
        my_x = lax.axis_index("x")
        my_y = lax.axis_index("y")
        my_z = lax.axis_index("z")
        ynbr = (my_x, 1 - my_y, my_z)
        xnbr = (1 - my_x, my_y, my_z)
        znbr = (my_x, my_y, 1 - my_z)

        k_me = 2 * my_x + my_z
        k_x = 2 * (1 - my_x) + my_z
        k_z = 2 * my_x + (1 - my_z)

        mine = my_y * m_per
        theirs = (1 - my_y) * m_per

        def rdma(offset, nbr, ssem, rsem):
            r = pltpu.make_async_remote_copy(
                src_ref=out_ref.at[pl.ds(offset, rows)],
                dst_ref=out_ref.at[pl.ds(offset, rows)],
                send_sem=ssem,
                recv_sem=rsem,
                device_id=nbr,
                device_id_type=pl.DeviceIdType.MESH,
            )
            r.start()
            return r

        barrier = pltpu.get_barrier_semaphore()
        pl.semaphore_signal(
            barrier, inc=1, device_id=ynbr,
            device_id_type=pl.DeviceIdType.MESH,
        )
        pl.semaphore_signal(
            xbar, inc=1, device_id=xnbr,
            device_id_type=pl.DeviceIdType.MESH,
        )
        pl.semaphore_signal(
            zbar, inc=1, device_id=znbr,
            device_id_type=pl.DeviceIdType.MESH,
        )

        out_ref[pl.ds(mine, m_per), :] = x_ref[...].astype(jnp.bfloat16)

        pl.semaphore_wait(barrier, 1)
        y_rdmas = [
            rdma(mine + k_me * q + c * rows, ynbr, ysend.at[c], yrecv.at[c])
            for c in range(CPQ)
        ]

        pl.semaphore_wait(xbar, 1)
        pl.semaphore_wait(zbar, 1)

        x2_rdmas, z2_rdmas = [], []
        for c in range(CPQ):
            y_rdmas[c].wait_recv()
            off = theirs + k_me * q + c * rows
            x2_rdmas.append(rdma(off, xnbr, x2send.at[c], x2recv.at[c]))
            z2_rdmas.append(rdma(off, znbr, z2send.at[c], z2recv.at[c]))

        x3_rdmas, z3_rdmas = [], []
        for c in range(CPQ // 2):
            z2_rdmas[c].wait_recv()
            x3_rdmas.append(
                rdma(theirs + k_z * q + c * rows, xnbr,
                     x3send.at[c], x3recv.at[c])
            )
        for c in range(CPQ // 2, CPQ):
            x2_rdmas[c].wait_recv()
            z3_rdmas.append(
                rdma(theirs + k_x * q + c * rows, znbr,
                     z3send.at[c - CPQ // 2], z3recv.at[c - CPQ // 2])
            )

        for c in range(CPQ // 2):
            x2_rdmas[c].wait_recv()
            z2_rdmas[CPQ // 2 + c].wait_recv()
            x3_rdmas[c].wait_recv()
            z3_rdmas[c].wait_recv()
        for c in range(CPQ):
            y_rdmas[c].wait_send()
            x2_rdmas[c].wait_send()
            z2_rdmas[c].wait_send()
        for c in range(CPQ // 2):
            x3_rdmas[c].wait_send()
            z3_rdmas[c].wait_send()

    return pl.pallas_call(
        body,
        out_shape=jax.ShapeDtypeStruct((2 * m_per, n), jnp.bfloat16),
        in_specs=[pl.BlockSpec(memory_space=pltpu.VMEM)],
        out_specs=pl.BlockSpec(memory_space=pltpu.VMEM),
        scratch_shapes=[
            pltpu.SemaphoreType.REGULAR,
            pltpu.SemaphoreType.REGULAR,
            pltpu.SemaphoreType.DMA((CPQ,)),
            pltpu.SemaphoreType.DMA((CPQ,)),
            pltpu.SemaphoreType.DMA((CPQ,)),
            pltpu.SemaphoreType.DMA((CPQ,)),
            pltpu.SemaphoreType.DMA((CPQ,)),
            pltpu.SemaphoreType.DMA((CPQ,)),
            pltpu.SemaphoreType.DMA((CPQ // 2,)),
            pltpu.SemaphoreType.DMA((CPQ // 2,)),
            pltpu.SemaphoreType.DMA((CPQ // 2,)),
            pltpu.SemaphoreType.DMA((CPQ // 2,)),
        ],
        compiler_params=pltpu.CompilerParams(collective_id=0),
    )(x)


# device time: 14324 ns/iter; 1.1086x vs baseline; 1.1086x over previous
import jax
import jax.numpy as jnp
from jax import lax
from jax.experimental import pallas as pl
from jax.experimental.pallas import tpu as pltpu

N_CHUNKS = 16
N_DIRECT = 10
N_FWD = 6


def kernel(x):
    m_per, n = x.shape
    rows = m_per // N_CHUNKS

    def body(x_ref, out_ref, zbar, ysend, yrecv, zsend, zrecv):
        my_x = lax.axis_index("x")
        my_y = lax.axis_index("y")
        my_z = lax.axis_index("z")
        ynbr = (my_x, 1 - my_y, my_z)
        znbr = (my_x, my_y, 1 - my_z)

        mine = my_y * m_per
        theirs = (1 - my_y) * m_per
        f_lo = N_DIRECT * my_z

        barrier = pltpu.get_barrier_semaphore()
        pl.semaphore_signal(
            barrier, inc=1, device_id=ynbr,
            device_id_type=pl.DeviceIdType.MESH,
        )
        pl.semaphore_signal(
            zbar, inc=1, device_id=znbr,
            device_id_type=pl.DeviceIdType.MESH,
        )

        out_ref[pl.ds(mine, m_per), :] = x_ref[...].astype(jnp.bfloat16)

        pl.semaphore_wait(barrier, 1)
        y_rdmas = []
        for i in range(N_DIRECT):
            g = f_lo + i if i < N_FWD else i
            off = mine + g * rows
            rdma = pltpu.make_async_remote_copy(
                src_ref=out_ref.at[pl.ds(off, rows)],
                dst_ref=out_ref.at[pl.ds(off, rows)],
                send_sem=ysend.at[i],
                recv_sem=yrecv.at[i],
                device_id=ynbr,
                device_id_type=pl.DeviceIdType.MESH,
            )
            rdma.start()
            y_rdmas.append(rdma)

        pl.semaphore_wait(zbar, 1)

        z_rdmas = []
        for i in range(N_FWD):
            y_rdmas[i].wait_recv()
            off = theirs + (f_lo + i) * rows
            rdma = pltpu.make_async_remote_copy(
                src_ref=out_ref.at[pl.ds(off, rows)],
                dst_ref=out_ref.at[pl.ds(off, rows)],
                send_sem=zsend.at[i],
                recv_sem=zrecv.at[i],
                device_id=znbr,
                device_id_type=pl.DeviceIdType.MESH,
            )
            rdma.start()
            z_rdmas.append(rdma)

        for i in range(N_FWD, N_DIRECT):
            y_rdmas[i].wait_recv()
        for i in range(N_FWD):
            z_rdmas[i].wait_recv()
        for i in range(N_DIRECT):
            y_rdmas[i].wait_send()
        for i in range(N_FWD):
            z_rdmas[i].wait_send()

    return pl.pallas_call(
        body,
        out_shape=jax.ShapeDtypeStruct((2 * m_per, n), jnp.bfloat16),
        in_specs=[pl.BlockSpec(memory_space=pltpu.VMEM)],
        out_specs=pl.BlockSpec(memory_space=pltpu.VMEM),
        scratch_shapes=[
            pltpu.SemaphoreType.REGULAR,
            pltpu.SemaphoreType.DMA((N_DIRECT,)),
            pltpu.SemaphoreType.DMA((N_DIRECT,)),
            pltpu.SemaphoreType.DMA((N_FWD,)),
            pltpu.SemaphoreType.DMA((N_FWD,)),
        ],
        compiler_params=pltpu.CompilerParams(collective_id=0),
    )(x)
